# baseline (speedup 1.0000x reference)
import functools

import jax
import jax.numpy as jnp
from jax import lax
from jax.experimental import pallas as pl
from jax.experimental.pallas import tpu as pltpu, tpu_sc as plsc

_INFO = plsc.get_sparse_core_info()
_NC, _NS, _L = _INFO.num_cores, _INFO.num_subcores, _INFO.num_lanes
_NW = _NC * _NS
_TOTAL = 16384 * 100
_PER_W = _TOTAL // _NW
_ITERS = _PER_W // _L


def _make_conf_kernel():
    mesh = plsc.VectorSubcoreMesh(core_axis_name="c", subcore_axis_name="s")

    @functools.partial(
        pl.kernel,
        mesh=mesh,
        out_type=jax.ShapeDtypeStruct((_NW, 2, _L), jnp.int32),
        scratch_types=[
            pltpu.VMEM((_PER_W,), jnp.float32),
            pltpu.VMEM((_PER_W,), jnp.int32),
            pltpu.VMEM((2, _L), jnp.int32),
        ],
    )
    def conf(out_hbm, tgt_hbm, res_hbm, o_v, t_v, r_v):
        wid = lax.axis_index("s") * _NC + lax.axis_index("c")
        base = wid * _PER_W
        pltpu.sync_copy(out_hbm.at[pl.ds(base, _PER_W)], o_v)
        pltpu.sync_copy(tgt_hbm.at[pl.ds(base, _PER_W)], t_v)

        zeros = jnp.zeros((_L,), jnp.int32)

        def body(i, carry):
            acc_ac, acc_b = carry
            o = o_v[pl.ds(i * _L, _L)]
            t = t_v[pl.ds(i * _L, _L)]
            p = o > 0.0
            acc_ac = acc_ac + jnp.where(p, t + (1 << 16), zeros)
            acc_b = acc_b + t
            return (acc_ac, acc_b)

        acc_ac, acc_b = lax.fori_loop(
            0, _ITERS, body, (zeros, zeros), unroll=4
        )
        r_v[0, :] = acc_ac
        r_v[1, :] = acc_b
        pltpu.sync_copy(r_v, res_hbm.at[wid])

    return conf


_conf = _make_conf_kernel()


def kernel(output, target):
    res = _conf(output.reshape(-1), target.reshape(-1))
    a = jnp.sum(res[:, 0, :] >> 16)
    c = jnp.sum(res[:, 0, :] & 0xFFFF)
    b = jnp.sum(res[:, 1, :])
    tp = c
    fp = a - c
    fn = _TOTAL - b - a + c
    return (tp, fp, fn)


# speedup vs baseline: 1.0029x; 1.0029x over previous
import functools

import jax
import jax.numpy as jnp
from jax import lax
from jax.experimental import pallas as pl
from jax.experimental.pallas import tpu as pltpu, tpu_sc as plsc

_INFO = plsc.get_sparse_core_info()
_NC, _NS, _L = _INFO.num_cores, _INFO.num_subcores, _INFO.num_lanes
_NW = _NC * _NS
_TOTAL = 16384 * 100
_PER_W = _TOTAL // _NW
_ITERS = _PER_W // _L


def _make_conf_kernel():
    mesh = plsc.VectorSubcoreMesh(core_axis_name="c", subcore_axis_name="s")

    @functools.partial(
        pl.kernel,
        mesh=mesh,
        out_type=jax.ShapeDtypeStruct((_NW, 2, _L), jnp.int32),
        scratch_types=[
            pltpu.VMEM((_PER_W,), jnp.float32),
            pltpu.VMEM((_PER_W,), jnp.int32),
            pltpu.VMEM((2, _L), jnp.int32),
        ],
    )
    def conf(out_hbm, tgt_hbm, res_hbm, o_v, t_v, r_v):
        wid = lax.axis_index("s") * _NC + lax.axis_index("c")
        base = wid * _PER_W
        pltpu.sync_copy(out_hbm.at[pl.ds(base, _PER_W)], o_v)
        pltpu.sync_copy(tgt_hbm.at[pl.ds(base, _PER_W)], t_v)

        zeros = jnp.zeros((_L,), jnp.int32)
        uv = 8

        def body(i, carry):
            accs = list(carry)
            base = i * (uv * _L)
            for u in range(uv):
                o = o_v[pl.ds(base + u * _L, _L)]
                t = t_v[pl.ds(base + u * _L, _L)]
                p = o > 0.0
                acc_ac, acc_b = accs[2 * u], accs[2 * u + 1]
                accs[2 * u] = acc_ac + jnp.where(p, t + (1 << 16), zeros)
                accs[2 * u + 1] = acc_b + t
            return tuple(accs)

        init = tuple(zeros for _ in range(2 * uv))
        accs = lax.fori_loop(0, _ITERS // uv, body, init, unroll=2)
        acc_ac = accs[0]
        acc_b = accs[1]
        for u in range(1, uv):
            acc_ac = acc_ac + accs[2 * u]
            acc_b = acc_b + accs[2 * u + 1]
        r_v[0, :] = acc_ac
        r_v[1, :] = acc_b
        pltpu.sync_copy(r_v, res_hbm.at[wid])

    return conf


_conf = _make_conf_kernel()


def kernel(output, target):
    res = _conf(output.reshape(-1), target.reshape(-1))
    a = jnp.sum(res[:, 0, :] >> 16)
    c = jnp.sum(res[:, 0, :] & 0xFFFF)
    b = jnp.sum(res[:, 1, :])
    tp = c
    fp = a - c
    fn = _TOTAL - b - a + c
    return (tp, fp, fn)


# speedup vs baseline: 1.4579x; 1.4537x over previous
import functools

import jax
import jax.numpy as jnp
from jax import lax
from jax.experimental import pallas as pl
from jax.experimental.pallas import tpu as pltpu, tpu_sc as plsc

_ROWS = 16384
_COLS = 100
_TOTAL = _ROWS * _COLS
_L = 16
_NW = 32
_ROWS_W = _ROWS // _NW
_CH_ROWS = 128
_NCH = _ROWS_W // _CH_ROWS
_NBUF = 2
_FULL_VECS = _COLS // _L
_TAIL_OFF = _COLS - _L
_TAIL_NEW = _L - (_COLS - _FULL_VECS * _L)


def _make_conf_kernel():
    mesh = plsc.VectorSubcoreMesh(core_axis_name="c", subcore_axis_name="s")

    @functools.partial(
        pl.kernel,
        mesh=mesh,
        out_type=jax.ShapeDtypeStruct((_NW, 2, _L), jnp.int32),
        scratch_types=[
            pltpu.VMEM((_NBUF, _CH_ROWS, _COLS), jnp.float32),
            pltpu.VMEM((_NBUF, _CH_ROWS, _COLS), jnp.int32),
            pltpu.VMEM((2, _L), jnp.int32),
            pltpu.SemaphoreType.DMA,
            pltpu.SemaphoreType.DMA,
        ],
    )
    def conf(out_hbm, tgt_hbm, res_hbm, o_v, t_v, r_v, sem0, sem1):
        nc = lax.axis_index("c")
        ns = lax.axis_index("s")
        wid = ns * 2 + nc
        row0 = wid * _ROWS_W
        sems = (sem0, sem1)

        def start(buf, c):
            r = row0 + c * _CH_ROWS
            return (
                pltpu.async_copy(
                    out_hbm.at[pl.ds(r, _CH_ROWS), :], o_v.at[buf], sems[buf]
                ),
                pltpu.async_copy(
                    tgt_hbm.at[pl.ds(r, _CH_ROWS), :], t_v.at[buf], sems[buf]
                ),
            )

        zeros = jnp.zeros((_L,), jnp.int32)
        tail_ok = lax.iota(jnp.int32, _L) >= _TAIL_NEW

        handles = [None, None]
        handles[0] = start(0, 0)

        acc_ac = zeros
        acc_b = zeros
        for c in range(_NCH):
            buf = c % _NBUF
            if c + 1 < _NCH:
                handles[(c + 1) % _NBUF] = start((c + 1) % _NBUF, c + 1)
            ha, hb = handles[buf]
            ha.wait()
            hb.wait()

            def body(r, carry, buf=buf):
                a_ac, a_b = carry
                for j in range(_FULL_VECS):
                    o = o_v[buf, r, pl.ds(j * _L, _L)]
                    t = t_v[buf, r, pl.ds(j * _L, _L)]
                    p = o > 0.0
                    a_ac = a_ac + jnp.where(p, t + (1 << 16), zeros)
                    a_b = a_b + t
                o = o_v[buf, r, pl.ds(_TAIL_OFF, _L)]
                t = t_v[buf, r, pl.ds(_TAIL_OFF, _L)]
                p = jnp.logical_and(o > 0.0, tail_ok)
                a_ac = a_ac + jnp.where(p, t + (1 << 16), zeros)
                a_b = a_b + jnp.where(tail_ok, t, zeros)
                return (a_ac, a_b)

            acc_ac, acc_b = lax.fori_loop(
                0, _CH_ROWS, body, (acc_ac, acc_b), unroll=2
            )

        r_v[0, :] = acc_ac
        r_v[1, :] = acc_b
        pltpu.sync_copy(r_v, res_hbm.at[wid])

    return conf


_conf = _make_conf_kernel()


def kernel(output, target):
    res = _conf(output, target)
    a = jnp.sum(res[:, 0, :] >> 16)
    c = jnp.sum(res[:, 0, :] & 0xFFFF)
    b = jnp.sum(res[:, 1, :])
    tp = c
    fp = a - c
    fn = _TOTAL - b - a + c
    return (tp, fp, fn)
